# Initial kernel scaffold; baseline (speedup 1.0000x reference)
import jax
import jax.numpy as jnp
from jax.experimental import pallas as pl


def kernel(data, edge_idx, bd_mask, poly_mesh, lin_W, lin_b, W1, as1, ad1, W2, as2, ad2, W3, as3, ad3, W4, as4, ad4):
    raise NotImplementedError("write your pallas kernel here")


# speedup vs baseline: 2.2574x; 2.2574x over previous
import functools

import jax
import jax.numpy as jnp
from jax import lax
from jax.experimental import pallas as pl
from jax.experimental.pallas import tpu as pltpu
from jax.experimental.pallas import tpu_sc as plsc

HEADS = 6
DIM = 2
N = 10000
E = 160000
NPAD = 10240
ROWS_PER_TILE = NPAD // 16
K_E = 64
NCHUNK = E // K_E
BM = 1024
OC = 64

_SELU_SCALE = 1.0507009873554805
_SELU_ALPHA = 1.6732632423543772


def _selu(x):
  return _SELU_SCALE * jnp.where(x > 0, x, _SELU_ALPHA * (jnp.exp(x) - 1.0))


def _t1_body(x_ref, w_ref, b_ref, o_ref):
  y = jnp.dot(x_ref[...], w_ref[...], preferred_element_type=jnp.float32)
  y = y + b_ref[...]
  col = lax.broadcasted_iota(jnp.int32, y.shape, 1)
  o_ref[...] = jnp.where(col < DIM, y, _selu(y))


def _t1_matmul(xp, w_aug, b_aug):
  in_dim, out_dim = w_aug.shape
  return pl.pallas_call(
      _t1_body,
      grid=(NPAD // BM,),
      in_specs=[
          pl.BlockSpec((BM, in_dim), lambda i: (i, 0)),
          pl.BlockSpec((in_dim, out_dim), lambda i: (0, 0)),
          pl.BlockSpec((1, out_dim), lambda i: (0, 0)),
      ],
      out_specs=pl.BlockSpec((BM, out_dim), lambda i: (i, 0)),
      out_shape=jax.ShapeDtypeStruct((NPAD, out_dim), jnp.float32),
  )(xp, w_aug, b_aug)


def _h3_body(x_ref, w_ref, o_ref):
  o_ref[0] = jnp.dot(x_ref[...], w_ref[...], preferred_element_type=jnp.float32)


def _h3_matmul(xp, wh, ch, row_w):
  in_dim = xp.shape[1]
  return pl.pallas_call(
      _h3_body,
      grid=(NPAD // BM, ch),
      in_specs=[
          pl.BlockSpec((BM, in_dim), lambda i, j: (i, 0)),
          pl.BlockSpec((in_dim, row_w), lambda i, j: (0, j)),
      ],
      out_specs=pl.BlockSpec((1, BM, row_w), lambda i, j: (j, i, 0)),
      out_shape=jax.ShapeDtypeStruct((ch, NPAD, row_w), jnp.float32),
  )(xp, wh)


def _ws_body(w_ref, asrc_ref, adst_ref, o_ref, *, out_w):
  ts = w_ref[...] * asrc_ref[...]
  td = w_ref[...] * adst_ref[...]
  cols = []
  for i in range(HEADS):
    cols.append(jnp.sum(ts[:, i * out_w:(i + 1) * out_w], axis=1,
                        keepdims=True))
  cols.append(jnp.zeros((ts.shape[0], 10), jnp.float32))
  for i in range(HEADS):
    cols.append(jnp.sum(td[:, i * out_w:(i + 1) * out_w], axis=1,
                        keepdims=True))
  cols.append(jnp.zeros((ts.shape[0], 10), jnp.float32))
  o_ref[...] = jnp.concatenate(cols, axis=1)


def _ws_prep(w, a_src, a_dst, out_w):
  in_dim, ho = w.shape
  af_s = a_src.reshape(1, ho)
  af_d = a_dst.reshape(1, ho)
  bi = 128
  return pl.pallas_call(
      functools.partial(_ws_body, out_w=out_w),
      grid=(in_dim // bi,),
      in_specs=[
          pl.BlockSpec((bi, ho), lambda i: (i, 0)),
          pl.BlockSpec((1, ho), lambda i: (0, 0)),
          pl.BlockSpec((1, ho), lambda i: (0, 0)),
      ],
      out_specs=pl.BlockSpec((bi, 32), lambda i: (i, 0)),
      out_shape=jax.ShapeDtypeStruct((in_dim, 32), jnp.float32),
  )(w, af_s, af_d)


def _s_body(x_ref, ws_ref, ssrc_ref, sdst_ref, c_ref, mx_ref, *, nblk):
  i = pl.program_id(0)
  y = jnp.dot(x_ref[...], ws_ref[...], preferred_element_type=jnp.float32)
  z = jnp.zeros((y.shape[0], 112), jnp.float32)
  ssrc_ref[...] = jnp.concatenate([y[:, 0:16], z], axis=1)
  sdst_ref[...] = jnp.concatenate([y[:, 16:32], z], axis=1)
  ms = jnp.max(y[:, 0:16])
  md = jnp.max(y[:, 16:32])

  @pl.when(i == 0)
  def _():
    mx_ref[0] = ms
    mx_ref[1] = md

  mx_ref[0] = jnp.maximum(mx_ref[0], ms)
  mx_ref[1] = jnp.maximum(mx_ref[1], md)

  @pl.when(i == nblk - 1)
  def _():
    c_ref[0, 0] = jnp.maximum(mx_ref[0], 0.0) + jnp.maximum(mx_ref[1], 0.0)


def _s_matmul(xp, ws):
  in_dim = xp.shape[1]
  nblk = NPAD // BM
  return pl.pallas_call(
      functools.partial(_s_body, nblk=nblk),
      grid=(nblk,),
      in_specs=[
          pl.BlockSpec((BM, in_dim), lambda i: (i, 0)),
          pl.BlockSpec((in_dim, 32), lambda i: (0, 0)),
      ],
      out_specs=[
          pl.BlockSpec((BM, 128), lambda i: (i, 0)),
          pl.BlockSpec((BM, 128), lambda i: (i, 0)),
          pl.BlockSpec(block_shape=(1, 1), index_map=lambda i: (0, 0),
                       memory_space=pltpu.SMEM),
      ],
      out_shape=[
          jax.ShapeDtypeStruct((NPAD, 128), jnp.float32),
          jax.ShapeDtypeStruct((NPAD, 128), jnp.float32),
          jax.ShapeDtypeStruct((1, 1), jnp.float32),
      ],
      scratch_shapes=[pltpu.SMEM((2,), jnp.float32)],
  )(xp, ws)


def _dsum_body(d2_ref, o_ref):
  o_ref[...] = d2_ref[0] + d2_ref[1]


def _dsum(d2):
  return pl.pallas_call(
      _dsum_body,
      grid=(NPAD // BM,),
      in_specs=[pl.BlockSpec((2, BM, 128), lambda i: (0, i, 0))],
      out_specs=pl.BlockSpec((BM, 128), lambda i: (i, 0)),
      out_shape=jax.ShapeDtypeStruct((NPAD, 128), jnp.float32),
  )(d2)


def _blend_body(cacc_ref, prev_ref, bd_ref, o_ref):
  b = bd_ref[...]
  o_ref[...] = b * prev_ref[...] + (1.0 - b) * (cacc_ref[0] + cacc_ref[1])


def _blend(cacc2, c_prev, bd128):
  return pl.pallas_call(
      _blend_body,
      grid=(NPAD // BM,),
      in_specs=[
          pl.BlockSpec((2, BM, 128), lambda i: (0, i, 0)),
          pl.BlockSpec((BM, 128), lambda i: (i, 0)),
          pl.BlockSpec((BM, 128), lambda i: (i, 0)),
      ],
      out_specs=pl.BlockSpec((BM, 128), lambda i: (i, 0)),
      out_shape=jax.ShapeDtypeStruct((NPAD, 128), jnp.float32),
  )(cacc2, c_prev, bd128)


_MESH = plsc.VectorSubcoreMesh(core_axis_name="c", subcore_axis_name="s")
_Z16 = functools.partial(jnp.zeros, (16,), jnp.float32)

_NQ32_HI, _NQ32_LO, _NQ32_EXTRA = 79, 78, 4
_NQ16_HI, _NQ16_LO, _NQ16_EXTRA = 157, 156, 4


def _pass1_body(src_hbm, dst_hbm, ssrc_hbm, sdst_hbm, c_hbm,
                e_hbm, d_hbm,
                src_v, dst_v, gsrc, gdst, ev128, ev16, cv, dacc, sem):
  c = lax.axis_index("c")
  s = lax.axis_index("s")
  wid = s * 2 + c
  row0 = s * ROWS_PER_TILE

  for k in range(K_E):
    for v in range(8):
      ev128[k, pl.ds(v * 16, 16)] = _Z16()
  for g in range(ROWS_PER_TILE // K_E):
    pltpu.sync_copy(ev128, dacc.at[pl.ds(row0 + g * K_E, K_E)])
  plsc.subcore_barrier()

  pltpu.sync_copy(c_hbm, cv)

  nq = jnp.where(wid < _NQ32_EXTRA, _NQ32_HI, _NQ32_LO)
  base_chunk = wid * _NQ32_LO + jnp.minimum(wid, _NQ32_EXTRA)

  def body(q, carry):
    ebase = pl.multiple_of((base_chunk + q) * K_E, K_E)
    pltpu.sync_copy(src_hbm.at[pl.ds(ebase, K_E)], src_v)
    pltpu.sync_copy(dst_hbm.at[pl.ds(ebase, K_E)], dst_v)
    pltpu.async_copy(ssrc_hbm.at[src_v], gsrc, sem).wait()
    pltpu.async_copy(sdst_hbm.at[dst_v], gdst, sem).wait()
    cvec = cv[...]

    def ebody(k, kc):
      x = gsrc[k, pl.ds(0, 16)] + gdst[k, pl.ds(0, 16)]
      l = jnp.maximum(x, 0.2 * x)
      e = jnp.exp(l - cvec)
      ev128[k, pl.ds(0, 16)] = e
      ev16[k, :] = e
      return kc

    lax.fori_loop(0, K_E, ebody, 0)
    pltpu.sync_copy(ev16, e_hbm.at[pl.ds(ebase, K_E)])
    pltpu.sync_copy(ev128, dacc.at[dst_v], add=True)
    return carry

  lax.fori_loop(0, nq, body, 0)
  plsc.subcore_barrier()
  pltpu.sync_copy(dacc.at[pl.ds(row0, ROWS_PER_TILE)],
                  d_hbm.at[c].at[pl.ds(row0, ROWS_PER_TILE)])


def _pass1(src, dst, ssrc, sdst, c16):
  kern = pl.kernel(
      _pass1_body,
      out_type=[
          jax.ShapeDtypeStruct((E, 16), jnp.float32),
          jax.ShapeDtypeStruct((2, NPAD, 128), jnp.float32),
      ],
      mesh=_MESH,
      scratch_types=[
          pltpu.VMEM((K_E,), jnp.int32),
          pltpu.VMEM((K_E,), jnp.int32),
          pltpu.VMEM((K_E, 128), jnp.float32),
          pltpu.VMEM((K_E, 128), jnp.float32),
          pltpu.VMEM((K_E, 128), jnp.float32),
          pltpu.VMEM((K_E, 16), jnp.float32),
          pltpu.VMEM((16,), jnp.float32),
          pltpu.VMEM_SHARED((NPAD, 128), jnp.float32),
          pltpu.SemaphoreType.DMA,
      ],
  )
  return kern(src, dst, ssrc, sdst, c16)


def _alpha_body(dst_hbm, e_hbm, dc_hbm, alpha_hbm,
                dst_v, ev, db, sem):
  c = lax.axis_index("c")
  s = lax.axis_index("s")
  wid = s * 2 + c

  nq = jnp.where(wid < _NQ32_EXTRA, _NQ32_HI, _NQ32_LO)
  base_chunk = wid * _NQ32_LO + jnp.minimum(wid, _NQ32_EXTRA)

  def body(q, carry):
    ebase = pl.multiple_of((base_chunk + q) * K_E, K_E)
    pltpu.sync_copy(dst_hbm.at[pl.ds(ebase, K_E)], dst_v)
    pltpu.sync_copy(e_hbm.at[pl.ds(ebase, K_E)], ev)
    pltpu.async_copy(dc_hbm.at[dst_v], db, sem).wait()

    def kbody(k, kc):
      ev[k, :] = ev[k, :] / (db[k, pl.ds(0, 16)] + 1e-16)
      return kc
    lax.fori_loop(0, K_E, kbody, 0)
    pltpu.sync_copy(ev, alpha_hbm.at[pl.ds(ebase, K_E)])
    return carry

  lax.fori_loop(0, nq, body, 0)


def _alpha(dst, e_buf, dcomb):
  kern = pl.kernel(
      _alpha_body,
      out_type=jax.ShapeDtypeStruct((E, 16), jnp.float32),
      mesh=_MESH,
      scratch_types=[
          pltpu.VMEM((K_E,), jnp.int32),
          pltpu.VMEM((K_E, 16), jnp.float32),
          pltpu.VMEM((K_E, 128), jnp.float32),
          pltpu.SemaphoreType.DMA,
      ],
  )
  return kern(dst, e_buf, dcomb)


def _pass2_body(src_hbm, dst_hbm, alpha_hbm, h2_hbm,
                feat_hbm,
                src_v, dst_v, idxa, ev, hbuf, msg, acc, sem,
                *, ch_per_sc):
  c = lax.axis_index("c")
  s = lax.axis_index("s")
  row0 = s * ROWS_PER_TILE

  nq = jnp.where(s < _NQ16_EXTRA, _NQ16_HI, _NQ16_LO)
  base_chunk = s * _NQ16_LO + jnp.minimum(s, _NQ16_EXTRA)

  def zero_msg():
    for k in range(K_E):
      for v in range(8):
        msg[k, pl.ds(v * 16, 16)] = _Z16()

  for jj in range(ch_per_sc):
    zero_msg()
    for g in range(ROWS_PER_TILE // K_E):
      pltpu.sync_copy(msg, acc.at[pl.ds(row0 + g * K_E, K_E)])
    plsc.subcore_barrier()

    jglob = c * ch_per_sc + jj
    hoff = jglob * NPAD

    def body(q, carry):
      ebase = pl.multiple_of((base_chunk + q) * K_E, K_E)
      pltpu.sync_copy(src_hbm.at[pl.ds(ebase, K_E)], src_v)
      pltpu.sync_copy(dst_hbm.at[pl.ds(ebase, K_E)], dst_v)
      pltpu.sync_copy(alpha_hbm.at[pl.ds(ebase, K_E)], ev)

      def ia(k, kc):
        idxa[pl.ds(k * 16, 16)] = src_v[pl.ds(k * 16, 16)] + hoff
        return kc
      lax.fori_loop(0, K_E // 16, ia, 0)
      pltpu.async_copy(h2_hbm.at[idxa], hbuf, sem).wait()

      def kbody(k, kc):
        al = ev[k, :]
        a0 = al[0]
        a1 = al[1]
        a2 = al[2]
        a3 = al[3]
        a4 = al[4]
        a5 = al[5]
        for v in range(OC // 16):
          m = a0 * hbuf[k, pl.ds(v * 16, 16)]
          m = m + a1 * hbuf[k, pl.ds(OC + v * 16, 16)]
          m = m + a2 * hbuf[k, pl.ds(2 * OC + v * 16, 16)]
          m = m + a3 * hbuf[k, pl.ds(3 * OC + v * 16, 16)]
          m = m + a4 * hbuf[k, pl.ds(4 * OC + v * 16, 16)]
          m = m + a5 * hbuf[k, pl.ds(5 * OC + v * 16, 16)]
          msg[k, pl.ds(v * 16, 16)] = m
        return kc

      lax.fori_loop(0, K_E, kbody, 0)
      pltpu.sync_copy(msg, acc.at[dst_v], add=True)
      return carry

    lax.fori_loop(0, nq, body, 0)
    plsc.subcore_barrier()

    for g in range(ROWS_PER_TILE // K_E):
      r0 = row0 + g * K_E
      pltpu.sync_copy(acc.at[pl.ds(r0, K_E)], msg)

      def sbody(k, kc):
        for v in range(OC // 16):
          x = msg[k, pl.ds(v * 16, 16)] * (1.0 / HEADS)
          neg = (_SELU_SCALE * _SELU_ALPHA) * (jnp.exp(x) - 1.0)
          msg[k, pl.ds(v * 16, 16)] = jnp.where(
              x > 0, _SELU_SCALE * x, neg)
        return kc
      lax.fori_loop(0, K_E, sbody, 0)
      pltpu.sync_copy(msg, feat_hbm.at[jglob].at[pl.ds(r0, K_E)])
    plsc.subcore_barrier()


def _pass2(src, dst, alpha, h2, ch_per_sc):
  ch_total = 2 * ch_per_sc
  kern = pl.kernel(
      functools.partial(_pass2_body, ch_per_sc=ch_per_sc),
      out_type=jax.ShapeDtypeStruct((ch_total, NPAD, 128), jnp.float32),
      mesh=_MESH,
      scratch_types=[
          pltpu.VMEM((K_E,), jnp.int32),
          pltpu.VMEM((K_E,), jnp.int32),
          pltpu.VMEM((K_E,), jnp.int32),
          pltpu.VMEM((K_E, 16), jnp.float32),
          pltpu.VMEM((K_E, HEADS * OC), jnp.float32),
          pltpu.VMEM((K_E, 128), jnp.float32),
          pltpu.VMEM_SHARED((NPAD, 128), jnp.float32),
          pltpu.SemaphoreType.DMA,
      ],
  )
  return kern(src, dst, alpha, h2)


def _pass2c_body(src_hbm, dst_hbm, alpha_hbm, c_hbm,
                 cacc_hbm,
                 src_v, dst_v, ev, cb, cwb, cacc, sem):
  c = lax.axis_index("c")
  s = lax.axis_index("s")
  wid = s * 2 + c
  row0 = s * ROWS_PER_TILE

  for k in range(K_E):
    for v in range(8):
      cwb[k, pl.ds(v * 16, 16)] = _Z16()
  for g in range(ROWS_PER_TILE // K_E):
    pltpu.sync_copy(cwb, cacc.at[pl.ds(row0 + g * K_E, K_E)])
  plsc.subcore_barrier()

  nq = jnp.where(wid < _NQ32_EXTRA, _NQ32_HI, _NQ32_LO)
  base_chunk = wid * _NQ32_LO + jnp.minimum(wid, _NQ32_EXTRA)

  def body(q, carry):
    ebase = pl.multiple_of((base_chunk + q) * K_E, K_E)
    pltpu.sync_copy(src_hbm.at[pl.ds(ebase, K_E)], src_v)
    pltpu.sync_copy(dst_hbm.at[pl.ds(ebase, K_E)], dst_v)
    pltpu.sync_copy(alpha_hbm.at[pl.ds(ebase, K_E)], ev)
    pltpu.async_copy(c_hbm.at[src_v], cb, sem).wait()

    def kbody(k, kc):
      al = ev[k, :]
      aw = (al[0] + al[1] + al[2] + al[3] + al[4] + al[5]) * (1.0 / HEADS)
      cwb[k, pl.ds(0, 16)] = aw * cb[k, pl.ds(0, 16)]
      return kc

    lax.fori_loop(0, K_E, kbody, 0)
    pltpu.sync_copy(cwb, cacc.at[dst_v], add=True)
    return carry

  lax.fori_loop(0, nq, body, 0)
  plsc.subcore_barrier()
  pltpu.sync_copy(cacc.at[pl.ds(row0, ROWS_PER_TILE)],
                  cacc_hbm.at[c].at[pl.ds(row0, ROWS_PER_TILE)])


def _pass2c(src, dst, alpha, c_prev):
  kern = pl.kernel(
      _pass2c_body,
      out_type=jax.ShapeDtypeStruct((2, NPAD, 128), jnp.float32),
      mesh=_MESH,
      scratch_types=[
          pltpu.VMEM((K_E,), jnp.int32),
          pltpu.VMEM((K_E,), jnp.int32),
          pltpu.VMEM((K_E, 16), jnp.float32),
          pltpu.VMEM((K_E, 128), jnp.float32),
          pltpu.VMEM((K_E, 128), jnp.float32),
          pltpu.VMEM_SHARED((NPAD, 128), jnp.float32),
          pltpu.SemaphoreType.DMA,
      ],
  )
  return kern(src, dst, alpha, c_prev)


def _layer(xp, w, a_src, a_dst, out_w, ch_per_sc, src, dst,
           c_prev, bd128, do_feat):
  in_dim, ho = w.shape
  ws = _ws_prep(w, a_src, a_dst, out_w)
  ssrc, sdst, c_sc = _s_matmul(xp, ws)
  c16 = jnp.broadcast_to(c_sc.reshape(1), (16,))
  e_buf, d2 = _pass1(src, dst, ssrc, sdst, c16)
  dcomb = _dsum(d2)
  alpha = _alpha(dst, e_buf, dcomb)
  feat = None
  if do_feat:
    ch_total = 2 * ch_per_sc
    opad = ch_total * OC
    wr = w.reshape(in_dim, HEADS, out_w)
    wp = jnp.pad(wr, ((0, 0), (0, 0), (0, opad - out_w)))
    wh = wp.reshape(in_dim, HEADS, ch_total, OC).transpose(0, 2, 1, 3)
    wh = wh.reshape(in_dim, ch_total * HEADS * OC)
    h3 = _h3_matmul(xp, wh, ch_total, HEADS * OC)
    h2 = h3.reshape(ch_total * NPAD, HEADS * OC)
    feat3 = _pass2(src, dst, alpha, h2, ch_per_sc)
    feat = feat3[:, :, :OC].transpose(1, 0, 2).reshape(NPAD, ch_total * OC)
    feat = feat[:, :out_w]
  cacc2 = _pass2c(src, dst, alpha, c_prev)
  cout = _blend(cacc2, c_prev, bd128)
  return feat, cout


def kernel(data, edge_idx, bd_mask, poly_mesh, lin_W, lin_b,
           W1, as1, ad1, W2, as2, ad2, W3, as3, ad3, W4, as4, ad4):
  del poly_mesh
  src = edge_idx[0]
  dst = edge_idx[1]

  datap = jnp.pad(data, ((0, NPAD - N), (0, 0)))
  bd128 = jnp.pad(jnp.broadcast_to(bd_mask, (N, 128)),
                  ((0, NPAD - N), (0, 0)))
  c0 = jnp.pad(data[:, :DIM], ((0, NPAD - N), (0, 128 - DIM)))

  in_dim = data.shape[1]
  sel = jnp.zeros((in_dim, DIM), jnp.float32).at[0, 0].set(1.0).at[1, 1].set(1.0)
  w_aug = jnp.concatenate([sel, lin_W], axis=1)
  b_aug = jnp.concatenate([jnp.zeros((DIM,), jnp.float32), lin_b]).reshape(1, -1)
  t1 = _t1_matmul(datap, w_aug, b_aug)

  f1, c1 = _layer(t1, W1, as1, ad1, 508, 4, src, dst, c0, bd128, True)
  t2 = jnp.concatenate([c1[:, :DIM], c0[:, :DIM], f1], axis=1)

  f2, c2 = _layer(t2, W2, as2, ad2, 250, 2, src, dst, c1, bd128, True)
  t3 = jnp.concatenate([c2[:, :DIM], c1[:, :DIM], c0[:, :DIM], f2], axis=1)

  f3, c3 = _layer(t3, W3, as3, ad3, 120, 1, src, dst, c2, bd128, True)
  t4 = jnp.concatenate([c3[:, :DIM], c2[:, :DIM], c1[:, :DIM],
                        c0[:, :DIM], f3], axis=1)

  _, c4 = _layer(t4, W4, as4, ad4, 20, 0, src, dst, c3, bd128, False)

  return c4[:N, :DIM]


# speedup vs baseline: 2.3073x; 1.0221x over previous
import functools

import jax
import jax.numpy as jnp
from jax import lax
from jax.experimental import pallas as pl
from jax.experimental.pallas import tpu as pltpu
from jax.experimental.pallas import tpu_sc as plsc

HEADS = 6
DIM = 2
N = 10000
E = 160000
NPAD = 10240
ROWS_PER_TILE = NPAD // 16
K_E = 64
NCHUNK = E // K_E
BM = 1024
OC = 64

_SELU_SCALE = 1.0507009873554805
_SELU_ALPHA = 1.6732632423543772


def _selu(x):
  return _SELU_SCALE * jnp.where(x > 0, x, _SELU_ALPHA * (jnp.exp(x) - 1.0))


def _t1_body(x_ref, w_ref, b_ref, o_ref):
  y = jnp.dot(x_ref[...], w_ref[...], preferred_element_type=jnp.float32)
  y = y + b_ref[...]
  col = lax.broadcasted_iota(jnp.int32, y.shape, 1)
  o_ref[...] = jnp.where(col < DIM, y, _selu(y))


def _t1_matmul(xp, w_aug, b_aug):
  in_dim, out_dim = w_aug.shape
  return pl.pallas_call(
      _t1_body,
      grid=(NPAD // BM,),
      in_specs=[
          pl.BlockSpec((BM, in_dim), lambda i: (i, 0)),
          pl.BlockSpec((in_dim, out_dim), lambda i: (0, 0)),
          pl.BlockSpec((1, out_dim), lambda i: (0, 0)),
      ],
      out_specs=pl.BlockSpec((BM, out_dim), lambda i: (i, 0)),
      out_shape=jax.ShapeDtypeStruct((NPAD, out_dim), jnp.float32),
  )(xp, w_aug, b_aug)


def _h3_body(x_ref, w_ref, o_ref):
  o_ref[0] = jnp.dot(x_ref[...], w_ref[...], preferred_element_type=jnp.float32)


def _h3_matmul(xp, wh, ch, row_w):
  in_dim = xp.shape[1]
  return pl.pallas_call(
      _h3_body,
      grid=(NPAD // BM, ch),
      in_specs=[
          pl.BlockSpec((BM, in_dim), lambda i, j: (i, 0)),
          pl.BlockSpec((in_dim, row_w), lambda i, j: (0, j)),
      ],
      out_specs=pl.BlockSpec((1, BM, row_w), lambda i, j: (j, i, 0)),
      out_shape=jax.ShapeDtypeStruct((ch, NPAD, row_w), jnp.float32),
  )(xp, wh)


def _ws_body(w_ref, asrc_ref, adst_ref, o_ref, *, out_w):
  ts = w_ref[...] * asrc_ref[...]
  td = w_ref[...] * adst_ref[...]
  cols = []
  for i in range(HEADS):
    cols.append(jnp.sum(ts[:, i * out_w:(i + 1) * out_w], axis=1,
                        keepdims=True))
  cols.append(jnp.zeros((ts.shape[0], 10), jnp.float32))
  for i in range(HEADS):
    cols.append(jnp.sum(td[:, i * out_w:(i + 1) * out_w], axis=1,
                        keepdims=True))
  cols.append(jnp.zeros((ts.shape[0], 10), jnp.float32))
  o_ref[...] = jnp.concatenate(cols, axis=1)


def _ws_prep(w, a_src, a_dst, out_w):
  in_dim, ho = w.shape
  af_s = a_src.reshape(1, ho)
  af_d = a_dst.reshape(1, ho)
  bi = 128
  return pl.pallas_call(
      functools.partial(_ws_body, out_w=out_w),
      grid=(in_dim // bi,),
      in_specs=[
          pl.BlockSpec((bi, ho), lambda i: (i, 0)),
          pl.BlockSpec((1, ho), lambda i: (0, 0)),
          pl.BlockSpec((1, ho), lambda i: (0, 0)),
      ],
      out_specs=pl.BlockSpec((bi, 32), lambda i: (i, 0)),
      out_shape=jax.ShapeDtypeStruct((in_dim, 32), jnp.float32),
  )(w, af_s, af_d)


def _s_body(x_ref, ws_ref, ssrc_ref, sdst_ref, c_ref, mx_ref, *, nblk):
  i = pl.program_id(0)
  y = jnp.dot(x_ref[...], ws_ref[...], preferred_element_type=jnp.float32)
  z = jnp.zeros((y.shape[0], 112), jnp.float32)
  ssrc_ref[...] = jnp.concatenate([y[:, 0:16], z], axis=1)
  sdst_ref[...] = jnp.concatenate([y[:, 16:32], z], axis=1)
  ms = jnp.max(y[:, 0:16])
  md = jnp.max(y[:, 16:32])

  @pl.when(i == 0)
  def _():
    mx_ref[0] = ms
    mx_ref[1] = md

  mx_ref[0] = jnp.maximum(mx_ref[0], ms)
  mx_ref[1] = jnp.maximum(mx_ref[1], md)

  @pl.when(i == nblk - 1)
  def _():
    c_ref[0, 0] = jnp.maximum(mx_ref[0], 0.0) + jnp.maximum(mx_ref[1], 0.0)


def _s_matmul(xp, ws):
  in_dim = xp.shape[1]
  nblk = NPAD // BM
  return pl.pallas_call(
      functools.partial(_s_body, nblk=nblk),
      grid=(nblk,),
      in_specs=[
          pl.BlockSpec((BM, in_dim), lambda i: (i, 0)),
          pl.BlockSpec((in_dim, 32), lambda i: (0, 0)),
      ],
      out_specs=[
          pl.BlockSpec((BM, 128), lambda i: (i, 0)),
          pl.BlockSpec((BM, 128), lambda i: (i, 0)),
          pl.BlockSpec(block_shape=(1, 1), index_map=lambda i: (0, 0),
                       memory_space=pltpu.SMEM),
      ],
      out_shape=[
          jax.ShapeDtypeStruct((NPAD, 128), jnp.float32),
          jax.ShapeDtypeStruct((NPAD, 128), jnp.float32),
          jax.ShapeDtypeStruct((1, 1), jnp.float32),
      ],
      scratch_shapes=[pltpu.SMEM((2,), jnp.float32)],
  )(xp, ws)


def _dsum_body(d2_ref, o_ref):
  o_ref[...] = d2_ref[0] + d2_ref[1]


def _dsum(d2):
  return pl.pallas_call(
      _dsum_body,
      grid=(NPAD // BM,),
      in_specs=[pl.BlockSpec((2, BM, 128), lambda i: (0, i, 0))],
      out_specs=pl.BlockSpec((BM, 128), lambda i: (i, 0)),
      out_shape=jax.ShapeDtypeStruct((NPAD, 128), jnp.float32),
  )(d2)


def _fsum_body(f2_ref, o_ref):
  o_ref[0] = _selu((f2_ref[0, 0] + f2_ref[0, 1]) * (1.0 / HEADS))


def _fsum(feat4, ch_total):
  return pl.pallas_call(
      _fsum_body,
      grid=(NPAD // BM, ch_total),
      in_specs=[pl.BlockSpec((1, 2, BM, 128), lambda i, j: (j, 0, i, 0))],
      out_specs=pl.BlockSpec((1, BM, 128), lambda i, j: (j, i, 0)),
      out_shape=jax.ShapeDtypeStruct((ch_total, NPAD, 128), jnp.float32),
  )(feat4)


def _blend_body(cacc_ref, prev_ref, bd_ref, o_ref):
  b = bd_ref[...]
  o_ref[...] = b * prev_ref[...] + (1.0 - b) * (cacc_ref[0] + cacc_ref[1])


def _blend(cacc2, c_prev, bd128):
  return pl.pallas_call(
      _blend_body,
      grid=(NPAD // BM,),
      in_specs=[
          pl.BlockSpec((2, BM, 128), lambda i: (0, i, 0)),
          pl.BlockSpec((BM, 128), lambda i: (i, 0)),
          pl.BlockSpec((BM, 128), lambda i: (i, 0)),
      ],
      out_specs=pl.BlockSpec((BM, 128), lambda i: (i, 0)),
      out_shape=jax.ShapeDtypeStruct((NPAD, 128), jnp.float32),
  )(cacc2, c_prev, bd128)


_MESH = plsc.VectorSubcoreMesh(core_axis_name="c", subcore_axis_name="s")
_Z16 = functools.partial(jnp.zeros, (16,), jnp.float32)

_NQ32_HI, _NQ32_LO, _NQ32_EXTRA = 79, 78, 4
_NQ16_HI, _NQ16_LO, _NQ16_EXTRA = 157, 156, 4


def _pass1_body(src_hbm, dst_hbm, ssrc_hbm, sdst_hbm, c_hbm,
                e_hbm, d_hbm,
                src_v, dst_v, gsrc, gdst, ev128, ev16, cv, dacc, sem):
  c = lax.axis_index("c")
  s = lax.axis_index("s")
  wid = s * 2 + c
  row0 = s * ROWS_PER_TILE

  for k in range(K_E):
    for v in range(8):
      ev128[k, pl.ds(v * 16, 16)] = _Z16()
  for g in range(ROWS_PER_TILE // K_E):
    pltpu.sync_copy(ev128, dacc.at[pl.ds(row0 + g * K_E, K_E)])
  plsc.subcore_barrier()

  pltpu.sync_copy(c_hbm, cv)

  nq = jnp.where(wid < _NQ32_EXTRA, _NQ32_HI, _NQ32_LO)
  base_chunk = wid * _NQ32_LO + jnp.minimum(wid, _NQ32_EXTRA)

  def body(q, carry):
    ebase = pl.multiple_of((base_chunk + q) * K_E, K_E)
    pltpu.sync_copy(src_hbm.at[pl.ds(ebase, K_E)], src_v)
    pltpu.sync_copy(dst_hbm.at[pl.ds(ebase, K_E)], dst_v)
    pltpu.async_copy(ssrc_hbm.at[src_v], gsrc, sem).wait()
    pltpu.async_copy(sdst_hbm.at[dst_v], gdst, sem).wait()
    cvec = cv[...]

    def ebody(k, kc):
      x = gsrc[k, pl.ds(0, 16)] + gdst[k, pl.ds(0, 16)]
      l = jnp.maximum(x, 0.2 * x)
      e = jnp.exp(l - cvec)
      ev128[k, pl.ds(0, 16)] = e
      ev16[k, :] = e
      return kc

    lax.fori_loop(0, K_E, ebody, 0)
    pltpu.sync_copy(ev16, e_hbm.at[pl.ds(ebase, K_E)])
    pltpu.sync_copy(ev128, dacc.at[dst_v], add=True)
    return carry

  lax.fori_loop(0, nq, body, 0)
  plsc.subcore_barrier()
  pltpu.sync_copy(dacc.at[pl.ds(row0, ROWS_PER_TILE)],
                  d_hbm.at[c].at[pl.ds(row0, ROWS_PER_TILE)])


def _pass1(src, dst, ssrc, sdst, c16):
  kern = pl.kernel(
      _pass1_body,
      out_type=[
          jax.ShapeDtypeStruct((E, 16), jnp.float32),
          jax.ShapeDtypeStruct((2, NPAD, 128), jnp.float32),
      ],
      mesh=_MESH,
      scratch_types=[
          pltpu.VMEM((K_E,), jnp.int32),
          pltpu.VMEM((K_E,), jnp.int32),
          pltpu.VMEM((K_E, 128), jnp.float32),
          pltpu.VMEM((K_E, 128), jnp.float32),
          pltpu.VMEM((K_E, 128), jnp.float32),
          pltpu.VMEM((K_E, 16), jnp.float32),
          pltpu.VMEM((16,), jnp.float32),
          pltpu.VMEM_SHARED((NPAD, 128), jnp.float32),
          pltpu.SemaphoreType.DMA,
      ],
  )
  return kern(src, dst, ssrc, sdst, c16)


def _alpha_body(src_hbm, dst_hbm, e_hbm, dc_hbm, c_hbm,
                alpha_hbm, cacc_hbm,
                src_v, dst_v, ev, db, cb, cwb, cacc, sem):
  c = lax.axis_index("c")
  s = lax.axis_index("s")
  wid = s * 2 + c
  row0 = s * ROWS_PER_TILE

  for k in range(K_E):
    for v in range(8):
      cwb[k, pl.ds(v * 16, 16)] = _Z16()
  for g in range(ROWS_PER_TILE // K_E):
    pltpu.sync_copy(cwb, cacc.at[pl.ds(row0 + g * K_E, K_E)])
  plsc.subcore_barrier()

  nq = jnp.where(wid < _NQ32_EXTRA, _NQ32_HI, _NQ32_LO)
  base_chunk = wid * _NQ32_LO + jnp.minimum(wid, _NQ32_EXTRA)

  def body(q, carry):
    ebase = pl.multiple_of((base_chunk + q) * K_E, K_E)
    pltpu.sync_copy(src_hbm.at[pl.ds(ebase, K_E)], src_v)
    pltpu.sync_copy(dst_hbm.at[pl.ds(ebase, K_E)], dst_v)
    pltpu.sync_copy(e_hbm.at[pl.ds(ebase, K_E)], ev)
    pltpu.async_copy(dc_hbm.at[dst_v], db, sem).wait()
    pltpu.async_copy(c_hbm.at[src_v], cb, sem).wait()

    def kbody(k, kc):
      al = ev[k, :] / (db[k, pl.ds(0, 16)] + 1e-16)
      ev[k, :] = al
      aw = (al[0] + al[1] + al[2] + al[3] + al[4] + al[5]) * (1.0 / HEADS)
      cwb[k, pl.ds(0, 16)] = aw * cb[k, pl.ds(0, 16)]
      return kc
    lax.fori_loop(0, K_E, kbody, 0)
    pltpu.sync_copy(ev, alpha_hbm.at[pl.ds(ebase, K_E)])
    pltpu.sync_copy(cwb, cacc.at[dst_v], add=True)
    return carry

  lax.fori_loop(0, nq, body, 0)
  plsc.subcore_barrier()
  pltpu.sync_copy(cacc.at[pl.ds(row0, ROWS_PER_TILE)],
                  cacc_hbm.at[c].at[pl.ds(row0, ROWS_PER_TILE)])


def _alpha_coord(src, dst, e_buf, dcomb, c_prev):
  kern = pl.kernel(
      _alpha_body,
      out_type=[
          jax.ShapeDtypeStruct((E, 16), jnp.float32),
          jax.ShapeDtypeStruct((2, NPAD, 128), jnp.float32),
      ],
      mesh=_MESH,
      scratch_types=[
          pltpu.VMEM((K_E,), jnp.int32),
          pltpu.VMEM((K_E,), jnp.int32),
          pltpu.VMEM((K_E, 16), jnp.float32),
          pltpu.VMEM((K_E, 128), jnp.float32),
          pltpu.VMEM((K_E, 128), jnp.float32),
          pltpu.VMEM((K_E, 128), jnp.float32),
          pltpu.VMEM_SHARED((NPAD, 128), jnp.float32),
          pltpu.SemaphoreType.DMA,
      ],
  )
  return kern(src, dst, e_buf, dcomb, c_prev)


def _pass2_body(src_hbm, dst_hbm, alpha_hbm, h2_hbm,
                feat_hbm,
                src_v, dst_v, idxa, ev, hbuf, msg, acc, sem,
                *, ch_per_sc):
  c = lax.axis_index("c")
  s = lax.axis_index("s")
  row0 = s * ROWS_PER_TILE

  nq = jnp.where(s < 2, 79, 78)
  base_chunk = c * (NCHUNK // 2) + s * 78 + jnp.minimum(s, 2)

  def zero_msg():
    for k in range(K_E):
      for v in range(8):
        msg[k, pl.ds(v * 16, 16)] = _Z16()

  for jj in range(ch_per_sc):
    zero_msg()
    for g in range(ROWS_PER_TILE // K_E):
      pltpu.sync_copy(msg, acc.at[pl.ds(row0 + g * K_E, K_E)])
    plsc.subcore_barrier()

    jglob = jj
    hoff = jglob * NPAD

    def body(q, carry):
      ebase = pl.multiple_of((base_chunk + q) * K_E, K_E)
      pltpu.sync_copy(src_hbm.at[pl.ds(ebase, K_E)], src_v)
      pltpu.sync_copy(dst_hbm.at[pl.ds(ebase, K_E)], dst_v)
      pltpu.sync_copy(alpha_hbm.at[pl.ds(ebase, K_E)], ev)

      def ia(k, kc):
        idxa[pl.ds(k * 16, 16)] = src_v[pl.ds(k * 16, 16)] + hoff
        return kc
      lax.fori_loop(0, K_E // 16, ia, 0)
      pltpu.async_copy(h2_hbm.at[idxa], hbuf, sem).wait()

      def kbody(k, kc):
        al = ev[k, :]
        a0 = al[0]
        a1 = al[1]
        a2 = al[2]
        a3 = al[3]
        a4 = al[4]
        a5 = al[5]
        for v in range(OC // 16):
          m = a0 * hbuf[k, pl.ds(v * 16, 16)]
          m = m + a1 * hbuf[k, pl.ds(OC + v * 16, 16)]
          m = m + a2 * hbuf[k, pl.ds(2 * OC + v * 16, 16)]
          m = m + a3 * hbuf[k, pl.ds(3 * OC + v * 16, 16)]
          m = m + a4 * hbuf[k, pl.ds(4 * OC + v * 16, 16)]
          m = m + a5 * hbuf[k, pl.ds(5 * OC + v * 16, 16)]
          msg[k, pl.ds(v * 16, 16)] = m
        return kc

      lax.fori_loop(0, K_E, kbody, 0)
      pltpu.sync_copy(msg, acc.at[dst_v], add=True)
      return carry

    lax.fori_loop(0, nq, body, 0)
    plsc.subcore_barrier()

    pltpu.sync_copy(acc.at[pl.ds(row0, ROWS_PER_TILE)],
                    feat_hbm.at[jglob].at[c].at[pl.ds(row0, ROWS_PER_TILE)])


def _pass2(src, dst, alpha, h2, ch_total):
  kern = pl.kernel(
      functools.partial(_pass2_body, ch_per_sc=ch_total),
      out_type=jax.ShapeDtypeStruct((ch_total, 2, NPAD, 128), jnp.float32),
      mesh=_MESH,
      scratch_types=[
          pltpu.VMEM((K_E,), jnp.int32),
          pltpu.VMEM((K_E,), jnp.int32),
          pltpu.VMEM((K_E,), jnp.int32),
          pltpu.VMEM((K_E, 16), jnp.float32),
          pltpu.VMEM((K_E, HEADS * OC), jnp.float32),
          pltpu.VMEM((K_E, 128), jnp.float32),
          pltpu.VMEM_SHARED((NPAD, 128), jnp.float32),
          pltpu.SemaphoreType.DMA,
      ],
  )
  return kern(src, dst, alpha, h2)


def _layer(xp, w, a_src, a_dst, out_w, ch_total, src, dst,
           c_prev, bd128, do_feat):
  in_dim, ho = w.shape
  ws = _ws_prep(w, a_src, a_dst, out_w)
  ssrc, sdst, c_sc = _s_matmul(xp, ws)
  c16 = jnp.broadcast_to(c_sc.reshape(1), (16,))
  e_buf, d2 = _pass1(src, dst, ssrc, sdst, c16)
  dcomb = _dsum(d2)
  alpha, cacc2 = _alpha_coord(src, dst, e_buf, dcomb, c_prev)
  feat = None
  if do_feat:
    opad = ch_total * OC
    wr = w.reshape(in_dim, HEADS, out_w)
    wp = jnp.pad(wr, ((0, 0), (0, 0), (0, opad - out_w)))
    wh = wp.reshape(in_dim, HEADS, ch_total, OC).transpose(0, 2, 1, 3)
    wh = wh.reshape(in_dim, ch_total * HEADS * OC)
    h3 = _h3_matmul(xp, wh, ch_total, HEADS * OC)
    h2 = h3.reshape(ch_total * NPAD, HEADS * OC)
    feat4 = _pass2(src, dst, alpha, h2, ch_total)
    feat3 = _fsum(feat4, ch_total)
    feat = feat3[:, :, :OC].transpose(1, 0, 2).reshape(NPAD, ch_total * OC)
    feat = feat[:, :out_w]
  cout = _blend(cacc2, c_prev, bd128)
  return feat, cout


def kernel(data, edge_idx, bd_mask, poly_mesh, lin_W, lin_b,
           W1, as1, ad1, W2, as2, ad2, W3, as3, ad3, W4, as4, ad4):
  del poly_mesh
  src = edge_idx[0]
  dst = edge_idx[1]

  datap = jnp.pad(data, ((0, NPAD - N), (0, 0)))
  bd128 = jnp.pad(jnp.broadcast_to(bd_mask, (N, 128)),
                  ((0, NPAD - N), (0, 0)))
  c0 = jnp.pad(data[:, :DIM], ((0, NPAD - N), (0, 128 - DIM)))

  in_dim = data.shape[1]
  sel = jnp.zeros((in_dim, DIM), jnp.float32).at[0, 0].set(1.0).at[1, 1].set(1.0)
  w_aug = jnp.concatenate([sel, lin_W], axis=1)
  b_aug = jnp.concatenate([jnp.zeros((DIM,), jnp.float32), lin_b]).reshape(1, -1)
  t1 = _t1_matmul(datap, w_aug, b_aug)

  f1, c1 = _layer(t1, W1, as1, ad1, 508, 8, src, dst, c0, bd128, True)
  t2 = jnp.concatenate([c1[:, :DIM], c0[:, :DIM], f1], axis=1)

  f2, c2 = _layer(t2, W2, as2, ad2, 250, 4, src, dst, c1, bd128, True)
  t3 = jnp.concatenate([c2[:, :DIM], c1[:, :DIM], c0[:, :DIM], f2], axis=1)

  f3, c3 = _layer(t3, W3, as3, ad3, 120, 2, src, dst, c2, bd128, True)
  t4 = jnp.concatenate([c3[:, :DIM], c2[:, :DIM], c1[:, :DIM],
                        c0[:, :DIM], f3], axis=1)

  _, c4 = _layer(t4, W4, as4, ad4, 20, 0, src, dst, c3, bd128, False)

  return c4[:N, :DIM]


# speedup vs baseline: 2.7573x; 1.1951x over previous
import functools

import jax
import jax.numpy as jnp
from jax import lax
from jax.experimental import pallas as pl
from jax.experimental.pallas import tpu as pltpu
from jax.experimental.pallas import tpu_sc as plsc

HEADS = 6
DIM = 2
N = 10000
E = 160000
NPAD = 10240
ROWS_PER_TILE = NPAD // 16
K_E = 64
NCHUNK = E // K_E
BM = 1024
OC = 64

_SELU_SCALE = 1.0507009873554805
_SELU_ALPHA = 1.6732632423543772


def _selu(x):
  return _SELU_SCALE * jnp.where(x > 0, x, _SELU_ALPHA * (jnp.exp(x) - 1.0))


def _t1_body(x_ref, w_ref, b_ref, o_ref):
  y = jnp.dot(x_ref[...], w_ref[...], preferred_element_type=jnp.float32)
  y = y + b_ref[...]
  col = lax.broadcasted_iota(jnp.int32, y.shape, 1)
  o_ref[...] = jnp.where(col < DIM, y, _selu(y))


def _t1_matmul(xp, w_aug, b_aug):
  in_dim, out_dim = w_aug.shape
  return pl.pallas_call(
      _t1_body,
      grid=(NPAD // BM,),
      in_specs=[
          pl.BlockSpec((BM, in_dim), lambda i: (i, 0)),
          pl.BlockSpec((in_dim, out_dim), lambda i: (0, 0)),
          pl.BlockSpec((1, out_dim), lambda i: (0, 0)),
      ],
      out_specs=pl.BlockSpec((BM, out_dim), lambda i: (i, 0)),
      out_shape=jax.ShapeDtypeStruct((NPAD, out_dim), jnp.float32),
  )(xp, w_aug, b_aug)


def _h3_body(x_ref, w_ref, o_ref):
  o_ref[0] = jnp.dot(x_ref[...], w_ref[...], preferred_element_type=jnp.float32)


def _h3_matmul(xp, wh, ch, row_w):
  in_dim = xp.shape[1]
  return pl.pallas_call(
      _h3_body,
      grid=(NPAD // BM, ch),
      in_specs=[
          pl.BlockSpec((BM, in_dim), lambda i, j: (i, 0)),
          pl.BlockSpec((in_dim, row_w), lambda i, j: (0, j)),
      ],
      out_specs=pl.BlockSpec((1, BM, row_w), lambda i, j: (j, i, 0)),
      out_shape=jax.ShapeDtypeStruct((ch, NPAD, row_w), jnp.float32),
  )(xp, wh)


def _ws_body(w_ref, asrc_ref, adst_ref, o_ref, *, out_w):
  ts = w_ref[...] * asrc_ref[...]
  td = w_ref[...] * adst_ref[...]
  cols = []
  for i in range(HEADS):
    cols.append(jnp.sum(ts[:, i * out_w:(i + 1) * out_w], axis=1,
                        keepdims=True))
  cols.append(jnp.zeros((ts.shape[0], 10), jnp.float32))
  for i in range(HEADS):
    cols.append(jnp.sum(td[:, i * out_w:(i + 1) * out_w], axis=1,
                        keepdims=True))
  cols.append(jnp.zeros((ts.shape[0], 10), jnp.float32))
  o_ref[...] = jnp.concatenate(cols, axis=1)


def _ws_prep(w, a_src, a_dst, out_w):
  in_dim, ho = w.shape
  af_s = a_src.reshape(1, ho)
  af_d = a_dst.reshape(1, ho)
  bi = 128
  return pl.pallas_call(
      functools.partial(_ws_body, out_w=out_w),
      grid=(in_dim // bi,),
      in_specs=[
          pl.BlockSpec((bi, ho), lambda i: (i, 0)),
          pl.BlockSpec((1, ho), lambda i: (0, 0)),
          pl.BlockSpec((1, ho), lambda i: (0, 0)),
      ],
      out_specs=pl.BlockSpec((bi, 32), lambda i: (i, 0)),
      out_shape=jax.ShapeDtypeStruct((in_dim, 32), jnp.float32),
  )(w, af_s, af_d)


def _s_body(x_ref, ws_ref, ssrc_ref, sdst_ref, c_ref, mx_ref, *, nblk):
  i = pl.program_id(0)
  y = jnp.dot(x_ref[...], ws_ref[...], preferred_element_type=jnp.float32)
  z = jnp.zeros((y.shape[0], 112), jnp.float32)
  ssrc_ref[...] = jnp.concatenate([y[:, 0:16], z], axis=1)
  sdst_ref[...] = jnp.concatenate([y[:, 16:32], z], axis=1)
  ms = jnp.max(y[:, 0:16])
  md = jnp.max(y[:, 16:32])

  @pl.when(i == 0)
  def _():
    mx_ref[0] = ms
    mx_ref[1] = md

  mx_ref[0] = jnp.maximum(mx_ref[0], ms)
  mx_ref[1] = jnp.maximum(mx_ref[1], md)

  @pl.when(i == nblk - 1)
  def _():
    c_ref[0, 0] = jnp.maximum(mx_ref[0], 0.0) + jnp.maximum(mx_ref[1], 0.0)


def _s_matmul(xp, ws):
  in_dim = xp.shape[1]
  nblk = NPAD // BM
  return pl.pallas_call(
      functools.partial(_s_body, nblk=nblk),
      grid=(nblk,),
      in_specs=[
          pl.BlockSpec((BM, in_dim), lambda i: (i, 0)),
          pl.BlockSpec((in_dim, 32), lambda i: (0, 0)),
      ],
      out_specs=[
          pl.BlockSpec((BM, 128), lambda i: (i, 0)),
          pl.BlockSpec((BM, 128), lambda i: (i, 0)),
          pl.BlockSpec(block_shape=(1, 1), index_map=lambda i: (0, 0),
                       memory_space=pltpu.SMEM),
      ],
      out_shape=[
          jax.ShapeDtypeStruct((NPAD, 128), jnp.float32),
          jax.ShapeDtypeStruct((NPAD, 128), jnp.float32),
          jax.ShapeDtypeStruct((1, 1), jnp.float32),
      ],
      scratch_shapes=[pltpu.SMEM((2,), jnp.float32)],
  )(xp, ws)


def _dsum_body(d2_ref, o_ref):
  o_ref[...] = d2_ref[0] + d2_ref[1]


def _dsum(d2):
  return pl.pallas_call(
      _dsum_body,
      grid=(NPAD // BM,),
      in_specs=[pl.BlockSpec((2, BM, 128), lambda i: (0, i, 0))],
      out_specs=pl.BlockSpec((BM, 128), lambda i: (i, 0)),
      out_shape=jax.ShapeDtypeStruct((NPAD, 128), jnp.float32),
  )(d2)


def _fsum_body(f2_ref, o_ref):
  o_ref[0] = _selu((f2_ref[0, 0] + f2_ref[0, 1]) * (1.0 / HEADS))


def _fsum(feat4, ch_total):
  return pl.pallas_call(
      _fsum_body,
      grid=(NPAD // BM, ch_total),
      in_specs=[pl.BlockSpec((1, 2, BM, 128), lambda i, j: (j, 0, i, 0))],
      out_specs=pl.BlockSpec((1, BM, 128), lambda i, j: (j, i, 0)),
      out_shape=jax.ShapeDtypeStruct((ch_total, NPAD, 128), jnp.float32),
  )(feat4)


def _blend_body(cacc_ref, prev_ref, bd_ref, o_ref):
  b = bd_ref[...]
  o_ref[...] = b * prev_ref[...] + (1.0 - b) * (cacc_ref[0] + cacc_ref[1])


def _blend(cacc2, c_prev, bd128):
  return pl.pallas_call(
      _blend_body,
      grid=(NPAD // BM,),
      in_specs=[
          pl.BlockSpec((2, BM, 128), lambda i: (0, i, 0)),
          pl.BlockSpec((BM, 128), lambda i: (i, 0)),
          pl.BlockSpec((BM, 128), lambda i: (i, 0)),
      ],
      out_specs=pl.BlockSpec((BM, 128), lambda i: (i, 0)),
      out_shape=jax.ShapeDtypeStruct((NPAD, 128), jnp.float32),
  )(cacc2, c_prev, bd128)


_MESH = plsc.VectorSubcoreMesh(core_axis_name="c", subcore_axis_name="s")
_Z16 = functools.partial(jnp.zeros, (16,), jnp.float32)

_NQ32_HI, _NQ32_LO, _NQ32_EXTRA = 79, 78, 4
_NQ16_HI, _NQ16_LO, _NQ16_EXTRA = 157, 156, 4


def _pass1_body(src_hbm, dst_hbm, ssrc_hbm, sdst_hbm, c_hbm,
                e_hbm, d_hbm,
                src_v, dst_v, gsrc, gdst, ev128, ev16, cv, dacc, sem):
  c = lax.axis_index("c")
  s = lax.axis_index("s")
  wid = s * 2 + c
  row0 = s * ROWS_PER_TILE

  for k in range(K_E):
    for v in range(8):
      ev128[k, pl.ds(v * 16, 16)] = _Z16()
  for g in range(ROWS_PER_TILE // K_E):
    pltpu.sync_copy(ev128, dacc.at[pl.ds(row0 + g * K_E, K_E)])
  plsc.subcore_barrier()

  pltpu.sync_copy(c_hbm, cv)

  nq = jnp.where(wid < _NQ32_EXTRA, _NQ32_HI, _NQ32_LO)
  base_chunk = wid * _NQ32_LO + jnp.minimum(wid, _NQ32_EXTRA)

  def body(q, carry):
    ebase = pl.multiple_of((base_chunk + q) * K_E, K_E)
    d1 = pltpu.async_copy(src_hbm.at[pl.ds(ebase, K_E)], src_v, sem)
    d2 = pltpu.async_copy(dst_hbm.at[pl.ds(ebase, K_E)], dst_v, sem)
    d1.wait()
    d2.wait()
    g1 = pltpu.async_copy(ssrc_hbm.at[src_v], gsrc, sem)
    g2 = pltpu.async_copy(sdst_hbm.at[dst_v], gdst, sem)
    g1.wait()
    g2.wait()
    cvec = cv[...]

    def ebody(k, kc):
      x = gsrc[k, pl.ds(0, 16)] + gdst[k, pl.ds(0, 16)]
      l = jnp.maximum(x, 0.2 * x)
      e = jnp.exp(l - cvec)
      ev128[k, pl.ds(0, 16)] = e
      ev16[k, :] = e
      return kc

    lax.fori_loop(0, K_E, ebody, 0)
    pltpu.sync_copy(ev16, e_hbm.at[pl.ds(ebase, K_E)])
    pltpu.sync_copy(ev128, dacc.at[dst_v], add=True)
    return carry

  lax.fori_loop(0, nq, body, 0)
  plsc.subcore_barrier()
  pltpu.sync_copy(dacc.at[pl.ds(row0, ROWS_PER_TILE)],
                  d_hbm.at[c].at[pl.ds(row0, ROWS_PER_TILE)])


def _pass1(src, dst, ssrc, sdst, c16):
  kern = pl.kernel(
      _pass1_body,
      out_type=[
          jax.ShapeDtypeStruct((E, 16), jnp.float32),
          jax.ShapeDtypeStruct((2, NPAD, 128), jnp.float32),
      ],
      mesh=_MESH,
      scratch_types=[
          pltpu.VMEM((K_E,), jnp.int32),
          pltpu.VMEM((K_E,), jnp.int32),
          pltpu.VMEM((K_E, 128), jnp.float32),
          pltpu.VMEM((K_E, 128), jnp.float32),
          pltpu.VMEM((K_E, 128), jnp.float32),
          pltpu.VMEM((K_E, 16), jnp.float32),
          pltpu.VMEM((16,), jnp.float32),
          pltpu.VMEM_SHARED((NPAD, 128), jnp.float32),
          pltpu.SemaphoreType.DMA,
      ],
  )
  return kern(src, dst, ssrc, sdst, c16)


def _alpha_body(src_hbm, dst_hbm, e_hbm, dc_hbm, c_hbm,
                alpha_hbm, cacc_hbm,
                src_v, dst_v, ev, db, cb, cwb, cacc, sem):
  c = lax.axis_index("c")
  s = lax.axis_index("s")
  wid = s * 2 + c
  row0 = s * ROWS_PER_TILE

  for k in range(K_E):
    for v in range(8):
      cwb[k, pl.ds(v * 16, 16)] = _Z16()
  for g in range(ROWS_PER_TILE // K_E):
    pltpu.sync_copy(cwb, cacc.at[pl.ds(row0 + g * K_E, K_E)])
  plsc.subcore_barrier()

  nq = jnp.where(wid < _NQ32_EXTRA, _NQ32_HI, _NQ32_LO)
  base_chunk = wid * _NQ32_LO + jnp.minimum(wid, _NQ32_EXTRA)

  def body(q, carry):
    ebase = pl.multiple_of((base_chunk + q) * K_E, K_E)
    d1 = pltpu.async_copy(src_hbm.at[pl.ds(ebase, K_E)], src_v, sem)
    d2 = pltpu.async_copy(dst_hbm.at[pl.ds(ebase, K_E)], dst_v, sem)
    d3 = pltpu.async_copy(e_hbm.at[pl.ds(ebase, K_E)], ev, sem)
    d1.wait()
    d2.wait()
    d3.wait()
    g1 = pltpu.async_copy(dc_hbm.at[dst_v], db, sem)
    g2 = pltpu.async_copy(c_hbm.at[src_v], cb, sem)
    g1.wait()
    g2.wait()

    def kbody(k, kc):
      al = ev[k, :] / (db[k, pl.ds(0, 16)] + 1e-16)
      ev[k, :] = al
      aw = (al[0] + al[1] + al[2] + al[3] + al[4] + al[5]) * (1.0 / HEADS)
      cwb[k, pl.ds(0, 16)] = aw * cb[k, pl.ds(0, 16)]
      return kc
    lax.fori_loop(0, K_E, kbody, 0)
    pltpu.sync_copy(ev, alpha_hbm.at[pl.ds(ebase, K_E)])
    pltpu.sync_copy(cwb, cacc.at[dst_v], add=True)
    return carry

  lax.fori_loop(0, nq, body, 0)
  plsc.subcore_barrier()
  pltpu.sync_copy(cacc.at[pl.ds(row0, ROWS_PER_TILE)],
                  cacc_hbm.at[c].at[pl.ds(row0, ROWS_PER_TILE)])


def _alpha_coord(src, dst, e_buf, dcomb, c_prev):
  kern = pl.kernel(
      _alpha_body,
      out_type=[
          jax.ShapeDtypeStruct((E, 16), jnp.float32),
          jax.ShapeDtypeStruct((2, NPAD, 128), jnp.float32),
      ],
      mesh=_MESH,
      scratch_types=[
          pltpu.VMEM((K_E,), jnp.int32),
          pltpu.VMEM((K_E,), jnp.int32),
          pltpu.VMEM((K_E, 16), jnp.float32),
          pltpu.VMEM((K_E, 128), jnp.float32),
          pltpu.VMEM((K_E, 128), jnp.float32),
          pltpu.VMEM((K_E, 128), jnp.float32),
          pltpu.VMEM_SHARED((NPAD, 128), jnp.float32),
          pltpu.SemaphoreType.DMA,
      ],
  )
  return kern(src, dst, e_buf, dcomb, c_prev)


def _pass2_body(src_hbm, dst_hbm, alpha_hbm, h2_hbm,
                feat_hbm,
                src_v, dst_v, idxa, ev, hbuf, msg, acc, sem,
                *, ch_per_sc):
  c = lax.axis_index("c")
  s = lax.axis_index("s")
  row0 = s * ROWS_PER_TILE

  nq = jnp.where(s < 2, 79, 78)
  base_chunk = c * (NCHUNK // 2) + s * 78 + jnp.minimum(s, 2)

  def zero_msg():
    for k in range(K_E):
      for v in range(8):
        msg[k, pl.ds(v * 16, 16)] = _Z16()

  for jj in range(ch_per_sc):
    zero_msg()
    for g in range(ROWS_PER_TILE // K_E):
      pltpu.sync_copy(msg, acc.at[pl.ds(row0 + g * K_E, K_E)])
    plsc.subcore_barrier()

    jglob = jj
    hoff = jglob * NPAD

    def body(q, carry):
      ebase = pl.multiple_of((base_chunk + q) * K_E, K_E)
      d1 = pltpu.async_copy(src_hbm.at[pl.ds(ebase, K_E)], src_v, sem)
      d2 = pltpu.async_copy(dst_hbm.at[pl.ds(ebase, K_E)], dst_v, sem)
      d3 = pltpu.async_copy(alpha_hbm.at[pl.ds(ebase, K_E)], ev, sem)
      d1.wait()
      d2.wait()
      d3.wait()

      def ia(k, kc):
        idxa[pl.ds(k * 16, 16)] = src_v[pl.ds(k * 16, 16)] + hoff
        return kc
      lax.fori_loop(0, K_E // 16, ia, 0)
      pltpu.async_copy(h2_hbm.at[idxa], hbuf, sem).wait()

      def kbody(k, kc):
        al = ev[k, :]
        a0 = al[0]
        a1 = al[1]
        a2 = al[2]
        a3 = al[3]
        a4 = al[4]
        a5 = al[5]
        for v in range(OC // 16):
          m = a0 * hbuf[k, pl.ds(v * 16, 16)]
          m = m + a1 * hbuf[k, pl.ds(OC + v * 16, 16)]
          m = m + a2 * hbuf[k, pl.ds(2 * OC + v * 16, 16)]
          m = m + a3 * hbuf[k, pl.ds(3 * OC + v * 16, 16)]
          m = m + a4 * hbuf[k, pl.ds(4 * OC + v * 16, 16)]
          m = m + a5 * hbuf[k, pl.ds(5 * OC + v * 16, 16)]
          msg[k, pl.ds(v * 16, 16)] = m
        return kc

      lax.fori_loop(0, K_E, kbody, 0)
      pltpu.sync_copy(msg, acc.at[dst_v], add=True)
      return carry

    lax.fori_loop(0, nq, body, 0)
    plsc.subcore_barrier()

    pltpu.sync_copy(acc.at[pl.ds(row0, ROWS_PER_TILE)],
                    feat_hbm.at[jglob].at[c].at[pl.ds(row0, ROWS_PER_TILE)])


def _pass2(src, dst, alpha, h2, ch_total):
  kern = pl.kernel(
      functools.partial(_pass2_body, ch_per_sc=ch_total),
      out_type=jax.ShapeDtypeStruct((ch_total, 2, NPAD, 128), jnp.float32),
      mesh=_MESH,
      scratch_types=[
          pltpu.VMEM((K_E,), jnp.int32),
          pltpu.VMEM((K_E,), jnp.int32),
          pltpu.VMEM((K_E,), jnp.int32),
          pltpu.VMEM((K_E, 16), jnp.float32),
          pltpu.VMEM((K_E, HEADS * OC), jnp.float32),
          pltpu.VMEM((K_E, 128), jnp.float32),
          pltpu.VMEM_SHARED((NPAD, 128), jnp.float32),
          pltpu.SemaphoreType.DMA,
      ],
  )
  return kern(src, dst, alpha, h2)


def _layer(xp, w, a_src, a_dst, out_w, ch_total, src, dst,
           c_prev, bd128, do_feat):
  in_dim, ho = w.shape
  ws = _ws_prep(w, a_src, a_dst, out_w)
  ssrc, sdst, c_sc = _s_matmul(xp, ws)
  c16 = jnp.broadcast_to(c_sc.reshape(1), (16,))
  e_buf, d2 = _pass1(src, dst, ssrc, sdst, c16)
  dcomb = _dsum(d2)
  alpha, cacc2 = _alpha_coord(src, dst, e_buf, dcomb, c_prev)
  feat = None
  if do_feat:
    opad = ch_total * OC
    wr = w.reshape(in_dim, HEADS, out_w)
    wp = jnp.pad(wr, ((0, 0), (0, 0), (0, opad - out_w)))
    wh = wp.reshape(in_dim, HEADS, ch_total, OC).transpose(0, 2, 1, 3)
    wh = wh.reshape(in_dim, ch_total * HEADS * OC)
    h3 = _h3_matmul(xp, wh, ch_total, HEADS * OC)
    h2 = h3.reshape(ch_total * NPAD, HEADS * OC)
    feat4 = _pass2(src, dst, alpha, h2, ch_total)
    feat3 = _fsum(feat4, ch_total)
    feat = feat3[:, :, :OC].transpose(1, 0, 2).reshape(NPAD, ch_total * OC)
    feat = feat[:, :out_w]
  cout = _blend(cacc2, c_prev, bd128)
  return feat, cout


def kernel(data, edge_idx, bd_mask, poly_mesh, lin_W, lin_b,
           W1, as1, ad1, W2, as2, ad2, W3, as3, ad3, W4, as4, ad4):
  del poly_mesh
  src = edge_idx[0]
  dst = edge_idx[1]

  datap = jnp.pad(data, ((0, NPAD - N), (0, 0)))
  bd128 = jnp.pad(jnp.broadcast_to(bd_mask, (N, 128)),
                  ((0, NPAD - N), (0, 0)))
  c0 = jnp.pad(data[:, :DIM], ((0, NPAD - N), (0, 128 - DIM)))

  in_dim = data.shape[1]
  sel = jnp.zeros((in_dim, DIM), jnp.float32).at[0, 0].set(1.0).at[1, 1].set(1.0)
  w_aug = jnp.concatenate([sel, lin_W], axis=1)
  b_aug = jnp.concatenate([jnp.zeros((DIM,), jnp.float32), lin_b]).reshape(1, -1)
  t1 = _t1_matmul(datap, w_aug, b_aug)

  f1, c1 = _layer(t1, W1, as1, ad1, 508, 8, src, dst, c0, bd128, True)
  t2 = jnp.concatenate([c1[:, :DIM], c0[:, :DIM], f1], axis=1)

  f2, c2 = _layer(t2, W2, as2, ad2, 250, 4, src, dst, c1, bd128, True)
  t3 = jnp.concatenate([c2[:, :DIM], c1[:, :DIM], c0[:, :DIM], f2], axis=1)

  f3, c3 = _layer(t3, W3, as3, ad3, 120, 2, src, dst, c2, bd128, True)
  t4 = jnp.concatenate([c3[:, :DIM], c2[:, :DIM], c1[:, :DIM],
                        c0[:, :DIM], f3], axis=1)

  _, c4 = _layer(t4, W4, as4, ad4, 20, 0, src, dst, c3, bd128, False)

  return c4[:N, :DIM]
